# baseline (speedup 1.0000x reference)
import functools

import numpy as np

import jax
import jax.numpy as jnp
from jax import lax
from jax.experimental import pallas as pl
from jax.experimental.pallas import tpu as pltpu
from jax.experimental.pallas import tpu_sc as plsc

_MASK_ABS = 0x7FFFFFFF
_HI_INIT = 0x7F800001
_SENTINEL = 0x7F700000

_BOUND = float(np.sqrt(2.0 / 6.0) * np.sqrt(3.0 / 2048.0))
_DELTA = 4.1e-5
_WLO = int(np.float32(_BOUND / 2 - _DELTA).view(np.int32))
_WHI = int(np.float32(_BOUND / 2 + _DELTA).view(np.int32))

_NTILES = 32
_LANES = 16
_WGRP = 16
_WPT = 32768
_CAP = 16384
_C0 = 8
_CAP_USE = _CAP - _C0 - _LANES


def _bits_of(scores):
    return lax.bitcast_convert_type(scores, jnp.int32) & _MASK_ABS


def _dyn_pivots(lo, hi):
    q = jnp.maximum((hi - lo) // 4, 1)
    return [lo + q, lo + 2 * q, lo + 3 * q]


def _bracket_update(lo, hi, pivots, counts, j):
    for p, c in zip(pivots, counts):
        below = c <= j
        lo = jnp.where(below, jnp.maximum(lo, p), lo)
        hi = jnp.where(below, hi, jnp.minimum(hi, p))
    return lo, hi


def _bitmap_body(scores_ref, pk_ref, cnts_ref, bm_ref, cs_ref, *, nblk):
    b = pl.program_id(0)

    @pl.when(b == 0)
    def _init():
        cs_ref[0] = 0
        cs_ref[1] = 0

    bits = _bits_of(scores_ref[...])
    cs_ref[0] += jnp.sum((bits < _WLO).astype(jnp.int32))
    cs_ref[1] += jnp.sum((bits < _WHI).astype(jnp.int32))
    inwin = (bits >= _WLO) & (bits < _WHI)
    words = jnp.dot(
        inwin.astype(jnp.bfloat16), pk_ref[...],
        preferred_element_type=jnp.float32,
    )
    bm_ref[...] = words.astype(jnp.int32)

    @pl.when(b == nblk - 1)
    def _emit():
        lane = lax.broadcasted_iota(jnp.int32, (1, 128), 1)
        cnts_ref[...] = jnp.where(
            lane == 0, cs_ref[0], jnp.where(lane == 1, cs_ref[1], 0)
        )


def _sc_extract_body(bm_hbm, rows_hbm, cand_hbm, tcnt_hbm,
                     words_v, nzidx_v, rows_v, cand_v, cnt_v, sem):
    wid = lax.axis_index("c") * 16 + lax.axis_index("s")
    wpt = bm_hbm.shape[1]
    base = wid * wpt
    pltpu.sync_copy(bm_hbm.at[wid], words_v)
    iota16 = lax.iota(jnp.int32, _LANES)

    safe = jnp.full((_LANES,), base, jnp.int32)

    def s1(g, ptr):
        w16 = words_v[pl.ds(g * _LANES, _LANES)]
        m = w16 != 0
        plsc.store_compressed(
            nzidx_v.at[pl.ds(ptr, _LANES)], base + g * _LANES + iota16, mask=m
        )
        return ptr + jnp.max(plsc.all_reduce_population_count(m))

    nz = lax.fori_loop(0, wpt // _LANES, s1, jnp.int32(0), unroll=4)

    nzf = nz // _LANES

    @pl.loop(0, 10)
    def _fill(g):
        off = (nzf + g) * _LANES
        pos = off + iota16
        cur = nzidx_v[pl.ds(off, _LANES)]
        nzidx_v[pl.ds(off, _LANES)] = jnp.where(pos >= nz, safe, cur)

    nchunks = (nz + 127) // 128

    def s2(ci, carry):
        cptr, tcnt = carry
        pltpu.async_copy(
            rows_hbm.at[nzidx_v.at[pl.ds(ci * 128, 128)]], rows_v, sem
        ).wait()

        def srow(v, cr):
            cptr2, tcnt2 = cr
            v16 = rows_v[v, pl.ds(0, _LANES)]
            bitsv = lax.bitcast_convert_type(v16, jnp.int32) & _MASK_ABS
            valid = (ci * 128 + v) < nz
            mv = (bitsv >= _WLO) & (bitsv < _WHI) & valid
            tcnt2 = tcnt2 + jnp.max(plsc.all_reduce_population_count(mv))
            ms = mv & (cptr2 < _CAP_USE)
            plsc.store_compressed(cand_v.at[pl.ds(cptr2, _LANES)], bitsv, mask=ms)
            cptr2 = cptr2 + jnp.max(plsc.all_reduce_population_count(ms))
            return cptr2, tcnt2

        return lax.fori_loop(0, 128, srow, (cptr, tcnt))

    cptr, tcnt = lax.fori_loop(
        0, nchunks, s2, (jnp.int32(_C0), jnp.int32(0))
    )

    cnt_v[...] = jnp.where(iota16 == 0, tcnt, jnp.where(iota16 == 1, cptr, 0))
    pltpu.sync_copy(cnt_v, tcnt_hbm.at[wid])
    pltpu.sync_copy(cand_v, cand_hbm.at[wid])


def _sc_extract(bitmap32, rows32):
    import dataclasses

    cp = pltpu.CompilerParams()
    if "needs_layout_passes" in pltpu.CompilerParams.__dataclass_fields__:
        cp = dataclasses.replace(cp, needs_layout_passes=False)
    if "use_tc_tiling_on_sc" in pltpu.CompilerParams.__dataclass_fields__:
        cp = dataclasses.replace(cp, use_tc_tiling_on_sc=False)
    mesh = plsc.VectorSubcoreMesh(core_axis_name="c", subcore_axis_name="s")
    kern = pl.kernel(
        _sc_extract_body,
        compiler_params=cp,
        out_type=[
            jax.ShapeDtypeStruct((_NTILES, _CAP), jnp.int32),
            jax.ShapeDtypeStruct((_NTILES, _LANES), jnp.int32),
        ],
        mesh=mesh,
        scratch_types=[
            pltpu.VMEM((_WPT,), jnp.int32),
            pltpu.VMEM((_WPT + 192,), jnp.int32),
            pltpu.VMEM((128, _WGRP), jnp.float32),
            pltpu.VMEM((_CAP,), jnp.int32),
            pltpu.VMEM((_LANES,), jnp.int32),
            pltpu.SemaphoreType.DMA,
        ],
    )
    return kern(bitmap32, rows32)


def _select_apply_body(cnts_ref, tcnt_ref, cand_ref, scores_ref, x_ref,
                       tout_ref, out_ref, st_ref, *, j):
    b = pl.program_id(0)

    @pl.when(b == 0)
    def _select():
        f_wlo = cnts_ref[0, 0]
        f_whi = cnts_ref[0, 1]
        tcnt = tcnt_ref[:, 0:1]
        stored = tcnt_ref[:, 1:2]
        col = lax.broadcasted_iota(jnp.int32, cand_ref.shape, 1)
        valid = (col >= _C0) & (col < stored)
        vals = jnp.where(valid, cand_ref[...], _SENTINEL)
        overflow = jnp.sum((tcnt != stored - _C0).astype(jnp.int32)) > 0

        def rbody(_, lohi):
            lo, hi = lohi
            pivots = _dyn_pivots(lo, hi)
            counts = [f_wlo + jnp.sum((vals < p).astype(jnp.int32)) for p in pivots]
            return _bracket_update(lo, hi, pivots, counts, j)

        lo, hi = lax.fori_loop(0, 12, rbody, (jnp.int32(_WLO), jnp.int32(_WHI)))
        ok = (f_wlo <= j) & (f_whi > j) & (hi - lo == 1) & ~overflow
        st_ref[0] = lo
        st_ref[1] = ok.astype(jnp.int32)
        lane = lax.broadcasted_iota(jnp.int32, (1, 128), 1)
        tout_ref[...] = jnp.where(
            lane == 0, lo, jnp.where(lane == 1, ok.astype(jnp.int32), 0)
        )

    @pl.when(b > 0)
    def _apply():
        mask = _bits_of(scores_ref[...]) >= st_ref[0]
        x = x_ref[...]
        out_ref[...] = jnp.where(mask[None, :, :], jnp.maximum(x, 0.0), x)


def _count_body(pivots_ref, scores_ref, counts_ref):
    i = pl.program_id(0)

    @pl.when(i == 0)
    def _init():
        counts_ref[...] = jnp.zeros_like(counts_ref)

    bits = _bits_of(scores_ref[...])
    lane = lax.broadcasted_iota(jnp.int32, (1, 128), 1)
    acc = jnp.zeros((1, 128), jnp.int32)
    for k in range(3):
        c = jnp.sum((bits < pivots_ref[k]).astype(jnp.int32))
        acc = acc + jnp.where(lane == k, c, 0)
    counts_ref[...] = counts_ref[...] + acc


def _apply_body(t_ref, scores_ref, x_ref, out_ref):
    mask = _bits_of(scores_ref[...]) >= t_ref[0]
    x = x_ref[...]
    out_ref[...] = jnp.where(mask[None, :, :], jnp.maximum(x, 0.0), x)


@jax.jit
def kernel(input, scores):
    batch, rows, cols = input.shape
    n = rows * cols
    j = n // 2
    block_rows = 128 if rows % 128 == 0 else rows
    nblk = rows // block_rows

    nwc = cols // _WGRP
    ci = np.arange(cols)[:, None]
    wi = np.arange(nwc)[None, :]
    packmat = jnp.asarray(
        np.where(ci // _WGRP == wi, 2.0 ** (ci % _WGRP), 0.0), jnp.bfloat16
    )

    cnts, bitmap = pl.pallas_call(
        functools.partial(_bitmap_body, nblk=nblk),
        grid=(nblk,),
        in_specs=[
            pl.BlockSpec((block_rows, cols), lambda b: (b, 0)),
            pl.BlockSpec((cols, nwc), lambda b: (0, 0)),
        ],
        out_specs=[
            pl.BlockSpec((1, 128), lambda b: (0, 0)),
            pl.BlockSpec((block_rows, nwc), lambda b: (b, 0)),
        ],
        out_shape=[
            jax.ShapeDtypeStruct((1, 128), jnp.int32),
            jax.ShapeDtypeStruct((rows, nwc), jnp.int32),
        ],
        scratch_shapes=[pltpu.SMEM((2,), jnp.int32)],
    )(scores, packmat)

    nwords = rows * cols // _WGRP
    cand, tcnt = _sc_extract(
        bitmap.reshape(_NTILES, nwords // _NTILES),
        scores.reshape(nwords, _WGRP),
    )

    tout, out0 = pl.pallas_call(
        functools.partial(_select_apply_body, j=j),
        grid=(nblk + 1,),
        in_specs=[
            pl.BlockSpec((1, 128), lambda b: (0, 0)),
            pl.BlockSpec((_NTILES, _LANES), lambda b: (0, 0)),
            pl.BlockSpec((_NTILES, _CAP), lambda b: (0, 0)),
            pl.BlockSpec((block_rows, cols), lambda b: (jnp.maximum(b - 1, 0), 0)),
            pl.BlockSpec(
                (batch, block_rows, cols),
                lambda b: (0, jnp.maximum(b - 1, 0), 0),
            ),
        ],
        out_specs=[
            pl.BlockSpec((1, 128), lambda b: (0, 0)),
            pl.BlockSpec(
                (batch, block_rows, cols),
                lambda b: (0, jnp.maximum(b - 1, 0), 0),
            ),
        ],
        out_shape=[
            jax.ShapeDtypeStruct((1, 128), jnp.int32),
            jax.ShapeDtypeStruct((batch, rows, cols), jnp.float32),
        ],
        scratch_shapes=[pltpu.SMEM((2,), jnp.int32)],
    )(cnts, tcnt, cand, scores, input)

    t_sel, ok = tout[0, 0], tout[0, 1] > 0
    lo0 = jnp.where(ok, t_sel, 0)
    hi0 = jnp.where(ok, t_sel + 1, jnp.int32(_HI_INIT))

    count_fn = pl.pallas_call(
        _count_body,
        grid=(nblk,),
        in_specs=[
            pl.BlockSpec(memory_space=pltpu.SMEM),
            pl.BlockSpec((block_rows, cols), lambda i: (i, 0)),
        ],
        out_specs=pl.BlockSpec((1, 128), lambda i: (0, 0)),
        out_shape=jax.ShapeDtypeStruct((1, 128), jnp.int32),
    )

    def w_cond(carry):
        lo, hi = carry
        return hi - lo > 1

    def w_body(carry):
        lo, hi = carry
        pivots = jnp.stack(_dyn_pivots(lo, hi))
        counts = count_fn(pivots, scores)[0, :3]
        return _bracket_update(lo, hi, list(pivots), list(counts), j)

    def redo(_):
        t_bits, _unused = lax.while_loop(w_cond, w_body, (lo0, hi0))
        return pl.pallas_call(
            _apply_body,
            grid=(nblk,),
            in_specs=[
                pl.BlockSpec(memory_space=pltpu.SMEM),
                pl.BlockSpec((block_rows, cols), lambda i: (i, 0)),
                pl.BlockSpec((batch, block_rows, cols), lambda i: (0, i, 0)),
            ],
            out_specs=pl.BlockSpec((batch, block_rows, cols), lambda i: (0, i, 0)),
            out_shape=jax.ShapeDtypeStruct((batch, rows, cols), jnp.float32),
        )(t_bits[None], scores, input)

    return lax.cond(~ok, redo, lambda _: out0, None)


# speedup vs baseline: 1.0668x; 1.0668x over previous
import functools

import numpy as np

import jax
import jax.numpy as jnp
from jax import lax
from jax.experimental import pallas as pl
from jax.experimental.pallas import tpu as pltpu
from jax.experimental.pallas import tpu_sc as plsc

_MASK_ABS = 0x7FFFFFFF
_HI_INIT = 0x7F800001
_SENTINEL = 0x7F700000

_BOUND = float(np.sqrt(2.0 / 6.0) * np.sqrt(3.0 / 2048.0))
_DELTA = 4.1e-5
_WLO = int(np.float32(_BOUND / 2 - _DELTA).view(np.int32))
_WHI = int(np.float32(_BOUND / 2 + _DELTA).view(np.int32))

_NTILES = 32
_LANES = 16
_WGRP = 16
_WPT = 32768
_CAP = 16384
_C0 = 8
_CAP_USE = _CAP - _C0 - _LANES


def _bits_of(scores):
    return lax.bitcast_convert_type(scores, jnp.int32) & _MASK_ABS


def _dyn_pivots(lo, hi):
    q = jnp.maximum((hi - lo) // 4, 1)
    return [lo + q, lo + 2 * q, lo + 3 * q]


def _bracket_update(lo, hi, pivots, counts, j):
    for p, c in zip(pivots, counts):
        below = c <= j
        lo = jnp.where(below, jnp.maximum(lo, p), lo)
        hi = jnp.where(below, hi, jnp.minimum(hi, p))
    return lo, hi


def _bitmap_body(scores_ref, pk_ref, cnts_ref, bm_ref, cs_ref, *, nblk):
    b = pl.program_id(0)

    @pl.when(b == 0)
    def _init():
        cs_ref[0] = 0
        cs_ref[1] = 0

    bits = _bits_of(scores_ref[...])
    cs_ref[0] += jnp.sum((bits < _WLO).astype(jnp.int32))
    cs_ref[1] += jnp.sum((bits < _WHI).astype(jnp.int32))
    inwin = (bits >= _WLO) & (bits < _WHI)
    words = jnp.dot(
        inwin.astype(jnp.bfloat16), pk_ref[...],
        preferred_element_type=jnp.float32,
    )
    bm_ref[...] = words.astype(jnp.int32)

    @pl.when(b == nblk - 1)
    def _emit():
        lane = lax.broadcasted_iota(jnp.int32, (1, 128), 1)
        cnts_ref[...] = jnp.where(
            lane == 0, cs_ref[0], jnp.where(lane == 1, cs_ref[1], 0)
        )


def _sc_extract_body(bm_hbm, rows_hbm, cand_hbm, tcnt_hbm,
                     words_v, nzidx_v, rows_v, rows_v2, cand_v, cnt_v, sem):
    wid = lax.axis_index("c") * 16 + lax.axis_index("s")
    wpt = bm_hbm.shape[1]
    base = wid * wpt
    pltpu.sync_copy(bm_hbm.at[wid], words_v)
    iota16 = lax.iota(jnp.int32, _LANES)

    safe = jnp.full((_LANES,), base, jnp.int32)

    def s1(g, ptr):
        w16 = words_v[pl.ds(g * _LANES, _LANES)]
        m = w16 != 0
        plsc.store_compressed(
            nzidx_v.at[pl.ds(ptr, _LANES)], base + g * _LANES + iota16, mask=m
        )
        return ptr + jnp.max(plsc.all_reduce_population_count(m))

    nz = lax.fori_loop(0, wpt // _LANES, s1, jnp.int32(0), unroll=4)

    nzf = nz // _LANES

    @pl.loop(0, 10)
    def _fill(g):
        off = (nzf + g) * _LANES
        pos = off + iota16
        cur = nzidx_v[pl.ds(off, _LANES)]
        nzidx_v[pl.ds(off, _LANES)] = jnp.where(pos >= nz, safe, cur)

    nchunks = (nz + 127) // 128

    def gather(ci, buf):
        return pltpu.make_async_copy(
            rows_hbm.at[nzidx_v.at[pl.ds(ci * 128, 128)]], buf, sem
        )

    @pl.when(nchunks > 0)
    def _prime():
        gather(0, rows_v).start()

    def filt(ci, buf, cptr):
        def srow(v, cptr2):
            v16 = buf[v, pl.ds(0, _LANES)]
            bitsv = lax.bitcast_convert_type(v16, jnp.int32) & _MASK_ABS
            valid = (ci * 128 + v) < nz
            ms = (bitsv >= _WLO) & (bitsv < _WHI) & valid & (cptr2 < _CAP_USE)
            plsc.store_compressed(cand_v.at[pl.ds(cptr2, _LANES)], bitsv, mask=ms)
            return cptr2 + jnp.max(plsc.all_reduce_population_count(ms))

        return lax.fori_loop(0, 128, srow, cptr)

    def s2(pi, cptr):
        ci0 = 2 * pi
        ci1 = ci0 + 1
        gather(ci0, rows_v).wait()

        @pl.when(ci1 < nchunks)
        def _nx1():
            gather(ci1, rows_v2).start()

        cptr = filt(ci0, rows_v, cptr)

        def odd(cptr):
            gather(ci1, rows_v2).wait()

            @pl.when(ci0 + 2 < nchunks)
            def _nx2():
                gather(ci0 + 2, rows_v).start()

            return filt(ci1, rows_v2, cptr)

        return lax.cond(ci1 < nchunks, odd, lambda c: c, cptr)

    cptr = lax.fori_loop(0, (nchunks + 1) // 2, s2, jnp.int32(_C0))

    cnt_v[...] = jnp.where(iota16 == 0, cptr, 0)
    pltpu.sync_copy(cnt_v, tcnt_hbm.at[wid])
    pltpu.sync_copy(cand_v, cand_hbm.at[wid])


def _sc_extract(bitmap32, rows32):
    import dataclasses

    cp = pltpu.CompilerParams()
    if "needs_layout_passes" in pltpu.CompilerParams.__dataclass_fields__:
        cp = dataclasses.replace(cp, needs_layout_passes=False)
    if "use_tc_tiling_on_sc" in pltpu.CompilerParams.__dataclass_fields__:
        cp = dataclasses.replace(cp, use_tc_tiling_on_sc=False)
    mesh = plsc.VectorSubcoreMesh(core_axis_name="c", subcore_axis_name="s")
    kern = pl.kernel(
        _sc_extract_body,
        compiler_params=cp,
        out_type=[
            jax.ShapeDtypeStruct((_NTILES, _CAP), jnp.int32),
            jax.ShapeDtypeStruct((_NTILES, _LANES), jnp.int32),
        ],
        mesh=mesh,
        scratch_types=[
            pltpu.VMEM((_WPT,), jnp.int32),
            pltpu.VMEM((_WPT + 192,), jnp.int32),
            pltpu.VMEM((128, _WGRP), jnp.float32),
            pltpu.VMEM((128, _WGRP), jnp.float32),
            pltpu.VMEM((_CAP,), jnp.int32),
            pltpu.VMEM((_LANES,), jnp.int32),
            pltpu.SemaphoreType.DMA,
        ],
    )
    return kern(bitmap32, rows32)


def _select_apply_body(cnts_ref, tcnt_ref, cand_ref, scores_ref, x_ref,
                       tout_ref, out_ref, st_ref, *, j):
    b = pl.program_id(0)

    @pl.when(b == 0)
    def _select():
        f_wlo = cnts_ref[0, 0]
        f_whi = cnts_ref[0, 1]
        stored = tcnt_ref[:, 0:1]
        col = lax.broadcasted_iota(jnp.int32, cand_ref.shape, 1)
        valid = (col >= _C0) & (col < stored)
        vals = jnp.where(valid, cand_ref[...], _SENTINEL)
        overflow = jnp.sum((stored >= _CAP_USE).astype(jnp.int32)) > 0

        def rbody(_, lohi):
            lo, hi = lohi
            pivots = _dyn_pivots(lo, hi)
            counts = [f_wlo + jnp.sum((vals < p).astype(jnp.int32)) for p in pivots]
            return _bracket_update(lo, hi, pivots, counts, j)

        lo, hi = lax.fori_loop(0, 12, rbody, (jnp.int32(_WLO), jnp.int32(_WHI)))
        ok = (f_wlo <= j) & (f_whi > j) & (hi - lo == 1) & ~overflow
        st_ref[0] = lo
        st_ref[1] = ok.astype(jnp.int32)
        lane = lax.broadcasted_iota(jnp.int32, (1, 128), 1)
        tout_ref[...] = jnp.where(
            lane == 0, lo, jnp.where(lane == 1, ok.astype(jnp.int32), 0)
        )

    @pl.when(b > 0)
    def _apply():
        mask = _bits_of(scores_ref[...]) >= st_ref[0]
        x = x_ref[...]
        out_ref[...] = jnp.where(mask[None, :, :], jnp.maximum(x, 0.0), x)


def _count_body(pivots_ref, scores_ref, counts_ref):
    i = pl.program_id(0)

    @pl.when(i == 0)
    def _init():
        counts_ref[...] = jnp.zeros_like(counts_ref)

    bits = _bits_of(scores_ref[...])
    lane = lax.broadcasted_iota(jnp.int32, (1, 128), 1)
    acc = jnp.zeros((1, 128), jnp.int32)
    for k in range(3):
        c = jnp.sum((bits < pivots_ref[k]).astype(jnp.int32))
        acc = acc + jnp.where(lane == k, c, 0)
    counts_ref[...] = counts_ref[...] + acc


def _apply_body(t_ref, scores_ref, x_ref, out_ref):
    mask = _bits_of(scores_ref[...]) >= t_ref[0]
    x = x_ref[...]
    out_ref[...] = jnp.where(mask[None, :, :], jnp.maximum(x, 0.0), x)


@jax.jit
def kernel(input, scores):
    batch, rows, cols = input.shape
    n = rows * cols
    j = n // 2
    block_rows = 256 if rows % 256 == 0 else rows
    nblk = rows // block_rows

    nwc = cols // _WGRP
    ci = np.arange(cols)[:, None]
    wi = np.arange(nwc)[None, :]
    packmat = jnp.asarray(
        np.where(ci // _WGRP == wi, 2.0 ** (ci % _WGRP), 0.0), jnp.bfloat16
    )

    cnts, bitmap = pl.pallas_call(
        functools.partial(_bitmap_body, nblk=nblk),
        grid=(nblk,),
        in_specs=[
            pl.BlockSpec((block_rows, cols), lambda b: (b, 0)),
            pl.BlockSpec((cols, nwc), lambda b: (0, 0)),
        ],
        out_specs=[
            pl.BlockSpec((1, 128), lambda b: (0, 0)),
            pl.BlockSpec((block_rows, nwc), lambda b: (b, 0)),
        ],
        out_shape=[
            jax.ShapeDtypeStruct((1, 128), jnp.int32),
            jax.ShapeDtypeStruct((rows, nwc), jnp.int32),
        ],
        scratch_shapes=[pltpu.SMEM((2,), jnp.int32)],
    )(scores, packmat)

    nwords = rows * cols // _WGRP
    cand, tcnt = _sc_extract(
        bitmap.reshape(_NTILES, nwords // _NTILES),
        scores.reshape(nwords, _WGRP),
    )

    tout, out0 = pl.pallas_call(
        functools.partial(_select_apply_body, j=j),
        grid=(nblk + 1,),
        in_specs=[
            pl.BlockSpec((1, 128), lambda b: (0, 0)),
            pl.BlockSpec((_NTILES, _LANES), lambda b: (0, 0)),
            pl.BlockSpec((_NTILES, _CAP), lambda b: (0, 0)),
            pl.BlockSpec((block_rows, cols), lambda b: (jnp.maximum(b - 1, 0), 0)),
            pl.BlockSpec(
                (batch, block_rows, cols),
                lambda b: (0, jnp.maximum(b - 1, 0), 0),
            ),
        ],
        out_specs=[
            pl.BlockSpec((1, 128), lambda b: (0, 0)),
            pl.BlockSpec(
                (batch, block_rows, cols),
                lambda b: (0, jnp.maximum(b - 1, 0), 0),
            ),
        ],
        out_shape=[
            jax.ShapeDtypeStruct((1, 128), jnp.int32),
            jax.ShapeDtypeStruct((batch, rows, cols), jnp.float32),
        ],
        scratch_shapes=[pltpu.SMEM((2,), jnp.int32)],
    )(cnts, tcnt, cand, scores, input)

    t_sel, ok = tout[0, 0], tout[0, 1] > 0
    lo0 = jnp.where(ok, t_sel, 0)
    hi0 = jnp.where(ok, t_sel + 1, jnp.int32(_HI_INIT))

    count_fn = pl.pallas_call(
        _count_body,
        grid=(nblk,),
        in_specs=[
            pl.BlockSpec(memory_space=pltpu.SMEM),
            pl.BlockSpec((block_rows, cols), lambda i: (i, 0)),
        ],
        out_specs=pl.BlockSpec((1, 128), lambda i: (0, 0)),
        out_shape=jax.ShapeDtypeStruct((1, 128), jnp.int32),
    )

    def w_cond(carry):
        lo, hi = carry
        return hi - lo > 1

    def w_body(carry):
        lo, hi = carry
        pivots = jnp.stack(_dyn_pivots(lo, hi))
        counts = count_fn(pivots, scores)[0, :3]
        return _bracket_update(lo, hi, list(pivots), list(counts), j)

    def redo(_):
        t_bits, _unused = lax.while_loop(w_cond, w_body, (lo0, hi0))
        return pl.pallas_call(
            _apply_body,
            grid=(nblk,),
            in_specs=[
                pl.BlockSpec(memory_space=pltpu.SMEM),
                pl.BlockSpec((block_rows, cols), lambda i: (i, 0)),
                pl.BlockSpec((batch, block_rows, cols), lambda i: (0, i, 0)),
            ],
            out_specs=pl.BlockSpec((batch, block_rows, cols), lambda i: (0, i, 0)),
            out_shape=jax.ShapeDtypeStruct((batch, rows, cols), jnp.float32),
        )(t_bits[None], scores, input)

    return lax.cond(~ok, redo, lambda _: out0, None)


# speedup vs baseline: 1.0929x; 1.0245x over previous
import functools

import numpy as np

import jax
import jax.numpy as jnp
from jax import lax
from jax.experimental import pallas as pl
from jax.experimental.pallas import tpu as pltpu
from jax.experimental.pallas import tpu_sc as plsc

_MASK_ABS = 0x7FFFFFFF
_HI_INIT = 0x7F800001
_SENTINEL = 0x7F700000

_BOUND = float(np.sqrt(2.0 / 6.0) * np.sqrt(3.0 / 2048.0))
_WLO = int(np.float32(_BOUND / 2).view(np.int32)) - 32768
_WHI = _WLO + 65535

_NTILES = 32
_LANES = 16
_WGRP = 16
_WPT = 32768
_CAP = 16384
_C0 = 8
_CAP_USE = _CAP - _C0 - _LANES


def _bits_of(scores):
    return lax.bitcast_convert_type(scores, jnp.int32) & _MASK_ABS


def _dyn_pivots(lo, hi):
    q = jnp.maximum((hi - lo) // 4, 1)
    return [lo + q, lo + 2 * q, lo + 3 * q]


def _bracket_update(lo, hi, pivots, counts, j):
    for p, c in zip(pivots, counts):
        below = c <= j
        lo = jnp.where(below, jnp.maximum(lo, p), lo)
        hi = jnp.where(below, hi, jnp.minimum(hi, p))
    return lo, hi


def _bitmap_body(scores_ref, pk_ref, cnts_ref, bm_ref, rel_ref, cs_ref, *, nblk):
    b = pl.program_id(0)

    @pl.when(b == 0)
    def _init():
        cs_ref[0] = 0
        cs_ref[1] = 0

    bits = _bits_of(scores_ref[...])
    cs_ref[0] += jnp.sum((bits < _WLO).astype(jnp.int32))
    cs_ref[1] += jnp.sum((bits < _WHI).astype(jnp.int32))
    inwin = (bits >= _WLO) & (bits < _WHI)
    words = jnp.dot(
        inwin.astype(jnp.bfloat16), pk_ref[...],
        preferred_element_type=jnp.float32,
    )
    bm_ref[...] = words.astype(jnp.int32)
    rel_ref[...] = jnp.clip(bits - _WLO, 0, 65535).astype(jnp.uint16)

    @pl.when(b == nblk - 1)
    def _emit():
        lane = lax.broadcasted_iota(jnp.int32, (1, 128), 1)
        cnts_ref[...] = jnp.where(
            lane == 0, cs_ref[0], jnp.where(lane == 1, cs_ref[1], 0)
        )


def _sc_extract_body(bm_hbm, rows_hbm, cand_hbm, tcnt_hbm,
                     words_v, nzidx_v, rows_v, rows_v2, cand_v, cnt_v, sem):
    wid = lax.axis_index("c") * 16 + lax.axis_index("s")
    wpt = bm_hbm.shape[1]
    base = wid * wpt
    pltpu.sync_copy(bm_hbm.at[wid], words_v)
    iota16 = lax.iota(jnp.int32, _LANES)

    safe = jnp.full((_LANES,), base, jnp.int32)

    def s1(g, ptr):
        w16 = words_v[pl.ds(g * _LANES, _LANES)]
        m = w16 != 0
        plsc.store_compressed(
            nzidx_v.at[pl.ds(ptr, _LANES)], base + g * _LANES + iota16, mask=m
        )
        return ptr + jnp.max(plsc.all_reduce_population_count(m))

    nz = lax.fori_loop(0, wpt // _LANES, s1, jnp.int32(0), unroll=4)

    nzf = nz // _LANES

    @pl.loop(0, 10)
    def _fill(g):
        off = (nzf + g) * _LANES
        pos = off + iota16
        cur = nzidx_v[pl.ds(off, _LANES)]
        nzidx_v[pl.ds(off, _LANES)] = jnp.where(pos >= nz, safe, cur)

    nchunks = (nz + 127) // 128

    def gather(ci, buf):
        return pltpu.make_async_copy(
            rows_hbm.at[nzidx_v.at[pl.ds(ci * 128, 128)]], buf, sem
        )

    @pl.when(nchunks > 0)
    def _prime():
        gather(0, rows_v).start()

    def filt(ci, buf, cptr):
        def srow(v, cptr2):
            v16 = buf[v, pl.ds(0, _LANES)]
            bitsv = lax.bitcast_convert_type(v16, jnp.int32) & _MASK_ABS
            valid = (ci * 128 + v) < nz
            ms = (bitsv >= _WLO) & (bitsv < _WHI) & valid & (cptr2 < _CAP_USE)
            plsc.store_compressed(cand_v.at[pl.ds(cptr2, _LANES)], bitsv, mask=ms)
            return cptr2 + jnp.max(plsc.all_reduce_population_count(ms))

        return lax.fori_loop(0, 128, srow, cptr)

    def s2(pi, cptr):
        ci0 = 2 * pi
        ci1 = ci0 + 1
        gather(ci0, rows_v).wait()

        @pl.when(ci1 < nchunks)
        def _nx1():
            gather(ci1, rows_v2).start()

        cptr = filt(ci0, rows_v, cptr)

        def odd(cptr):
            gather(ci1, rows_v2).wait()

            @pl.when(ci0 + 2 < nchunks)
            def _nx2():
                gather(ci0 + 2, rows_v).start()

            return filt(ci1, rows_v2, cptr)

        return lax.cond(ci1 < nchunks, odd, lambda c: c, cptr)

    cptr = lax.fori_loop(0, (nchunks + 1) // 2, s2, jnp.int32(_C0))

    cnt_v[...] = jnp.where(iota16 == 0, cptr, 0)
    pltpu.sync_copy(cnt_v, tcnt_hbm.at[wid])
    pltpu.sync_copy(cand_v, cand_hbm.at[wid])


def _sc_extract(bitmap32, rows32):
    import dataclasses

    cp = pltpu.CompilerParams()
    if "needs_layout_passes" in pltpu.CompilerParams.__dataclass_fields__:
        cp = dataclasses.replace(cp, needs_layout_passes=False)
    if "use_tc_tiling_on_sc" in pltpu.CompilerParams.__dataclass_fields__:
        cp = dataclasses.replace(cp, use_tc_tiling_on_sc=False)
    mesh = plsc.VectorSubcoreMesh(core_axis_name="c", subcore_axis_name="s")
    kern = pl.kernel(
        _sc_extract_body,
        compiler_params=cp,
        out_type=[
            jax.ShapeDtypeStruct((_NTILES, _CAP), jnp.int32),
            jax.ShapeDtypeStruct((_NTILES, _LANES), jnp.int32),
        ],
        mesh=mesh,
        scratch_types=[
            pltpu.VMEM((_WPT,), jnp.int32),
            pltpu.VMEM((_WPT + 192,), jnp.int32),
            pltpu.VMEM((128, _WGRP), jnp.float32),
            pltpu.VMEM((128, _WGRP), jnp.float32),
            pltpu.VMEM((_CAP,), jnp.int32),
            pltpu.VMEM((_LANES,), jnp.int32),
            pltpu.SemaphoreType.DMA,
        ],
    )
    return kern(bitmap32, rows32)


def _select_apply_body(cnts_ref, tcnt_ref, cand_ref, scores_ref, x_ref,
                       tout_ref, out_ref, st_ref, *, j):
    b = pl.program_id(0)

    @pl.when(b == 0)
    def _select():
        f_wlo = cnts_ref[0, 0]
        f_whi = cnts_ref[0, 1]
        stored = tcnt_ref[:, 0:1]
        col = lax.broadcasted_iota(jnp.int32, cand_ref.shape, 1)
        valid = (col >= _C0) & (col < stored)
        vals = jnp.where(valid, cand_ref[...], _SENTINEL)
        overflow = jnp.sum((stored >= _CAP_USE).astype(jnp.int32)) > 0

        def rbody(_, lohi):
            lo, hi = lohi
            pivots = _dyn_pivots(lo, hi)
            counts = [f_wlo + jnp.sum((vals < p).astype(jnp.int32)) for p in pivots]
            return _bracket_update(lo, hi, pivots, counts, j)

        lo, hi = lax.fori_loop(0, 12, rbody, (jnp.int32(_WLO), jnp.int32(_WHI)))
        ok = (f_wlo <= j) & (f_whi > j) & (hi - lo == 1) & ~overflow
        st_ref[0] = lo
        st_ref[1] = ok.astype(jnp.int32)
        lane = lax.broadcasted_iota(jnp.int32, (1, 128), 1)
        tout_ref[...] = jnp.where(
            lane == 0, lo, jnp.where(lane == 1, ok.astype(jnp.int32), 0)
        )

    @pl.when(b > 0)
    def _apply():
        mask = scores_ref[...].astype(jnp.int32) >= st_ref[0] - _WLO
        x = x_ref[...]
        out_ref[...] = jnp.where(mask[None, :, :], jnp.maximum(x, 0.0), x)


def _count_body(pivots_ref, scores_ref, counts_ref):
    i = pl.program_id(0)

    @pl.when(i == 0)
    def _init():
        counts_ref[...] = jnp.zeros_like(counts_ref)

    bits = _bits_of(scores_ref[...])
    lane = lax.broadcasted_iota(jnp.int32, (1, 128), 1)
    acc = jnp.zeros((1, 128), jnp.int32)
    for k in range(3):
        c = jnp.sum((bits < pivots_ref[k]).astype(jnp.int32))
        acc = acc + jnp.where(lane == k, c, 0)
    counts_ref[...] = counts_ref[...] + acc


def _apply_body(t_ref, scores_ref, x_ref, out_ref):
    mask = _bits_of(scores_ref[...]) >= t_ref[0]
    x = x_ref[...]
    out_ref[...] = jnp.where(mask[None, :, :], jnp.maximum(x, 0.0), x)


@jax.jit
def kernel(input, scores):
    batch, rows, cols = input.shape
    n = rows * cols
    j = n // 2
    block_rows = 256 if rows % 256 == 0 else rows
    nblk = rows // block_rows

    nwc = cols // _WGRP
    ci = np.arange(cols)[:, None]
    wi = np.arange(nwc)[None, :]
    packmat = jnp.asarray(
        np.where(ci // _WGRP == wi, 2.0 ** (ci % _WGRP), 0.0), jnp.bfloat16
    )

    cnts, bitmap, rel = pl.pallas_call(
        functools.partial(_bitmap_body, nblk=nblk),
        grid=(nblk,),
        in_specs=[
            pl.BlockSpec((block_rows, cols), lambda b: (b, 0)),
            pl.BlockSpec((cols, nwc), lambda b: (0, 0)),
        ],
        out_specs=[
            pl.BlockSpec((1, 128), lambda b: (0, 0)),
            pl.BlockSpec((block_rows, nwc), lambda b: (b, 0)),
            pl.BlockSpec((block_rows, cols), lambda b: (b, 0)),
        ],
        out_shape=[
            jax.ShapeDtypeStruct((1, 128), jnp.int32),
            jax.ShapeDtypeStruct((rows, nwc), jnp.int32),
            jax.ShapeDtypeStruct((rows, cols), jnp.uint16),
        ],
        scratch_shapes=[pltpu.SMEM((2,), jnp.int32)],
    )(scores, packmat)

    nwords = rows * cols // _WGRP
    cand, tcnt = _sc_extract(
        bitmap.reshape(_NTILES, nwords // _NTILES),
        scores.reshape(nwords, _WGRP),
    )

    tout, out0 = pl.pallas_call(
        functools.partial(_select_apply_body, j=j),
        grid=(nblk + 1,),
        in_specs=[
            pl.BlockSpec((1, 128), lambda b: (0, 0)),
            pl.BlockSpec((_NTILES, _LANES), lambda b: (0, 0)),
            pl.BlockSpec((_NTILES, _CAP), lambda b: (0, 0)),
            pl.BlockSpec((block_rows, cols), lambda b: (jnp.maximum(b - 1, 0), 0)),
            pl.BlockSpec(
                (batch, block_rows, cols),
                lambda b: (0, jnp.maximum(b - 1, 0), 0),
            ),
        ],
        out_specs=[
            pl.BlockSpec((1, 128), lambda b: (0, 0)),
            pl.BlockSpec(
                (batch, block_rows, cols),
                lambda b: (0, jnp.maximum(b - 1, 0), 0),
            ),
        ],
        out_shape=[
            jax.ShapeDtypeStruct((1, 128), jnp.int32),
            jax.ShapeDtypeStruct((batch, rows, cols), jnp.float32),
        ],
        scratch_shapes=[pltpu.SMEM((2,), jnp.int32)],
    )(cnts, tcnt, cand, rel, input)

    t_sel, ok = tout[0, 0], tout[0, 1] > 0
    lo0 = jnp.where(ok, t_sel, 0)
    hi0 = jnp.where(ok, t_sel + 1, jnp.int32(_HI_INIT))

    count_fn = pl.pallas_call(
        _count_body,
        grid=(nblk,),
        in_specs=[
            pl.BlockSpec(memory_space=pltpu.SMEM),
            pl.BlockSpec((block_rows, cols), lambda i: (i, 0)),
        ],
        out_specs=pl.BlockSpec((1, 128), lambda i: (0, 0)),
        out_shape=jax.ShapeDtypeStruct((1, 128), jnp.int32),
    )

    def w_cond(carry):
        lo, hi = carry
        return hi - lo > 1

    def w_body(carry):
        lo, hi = carry
        pivots = jnp.stack(_dyn_pivots(lo, hi))
        counts = count_fn(pivots, scores)[0, :3]
        return _bracket_update(lo, hi, list(pivots), list(counts), j)

    def redo(_):
        t_bits, _unused = lax.while_loop(w_cond, w_body, (lo0, hi0))
        return pl.pallas_call(
            _apply_body,
            grid=(nblk,),
            in_specs=[
                pl.BlockSpec(memory_space=pltpu.SMEM),
                pl.BlockSpec((block_rows, cols), lambda i: (i, 0)),
                pl.BlockSpec((batch, block_rows, cols), lambda i: (0, i, 0)),
            ],
            out_specs=pl.BlockSpec((batch, block_rows, cols), lambda i: (0, i, 0)),
            out_shape=jax.ShapeDtypeStruct((batch, rows, cols), jnp.float32),
        )(t_bits[None], scores, input)

    return lax.cond(~ok, redo, lambda _: out0, None)
